# Initial kernel scaffold; baseline (speedup 1.0000x reference)
import jax
import jax.numpy as jnp
from jax.experimental import pallas as pl


def kernel(node_data, edge_weights, W0, b0, W1, b1, bn0_g, bn0_b, bn1_g, bn1_b):
    raise NotImplementedError("write your pallas kernel here")


# speedup vs baseline: 1.5428x; 1.5428x over previous
import jax
import jax.numpy as jnp
from jax.experimental import pallas as pl

TILE1 = 256
TILE2 = 1024


def _k1_body(ew_ref, x_ref, avg_ref, stats_ref):
    t = pl.program_id(0)
    i = pl.program_id(1)

    @pl.when(jnp.logical_and(t == 0, i == 0))
    def _init():
        stats_ref[...] = jnp.zeros_like(stats_ref)

    ew = ew_ref[0]
    x = x_ref[0]
    s = jnp.dot(ew, x, preferred_element_type=jnp.float32)
    z = jnp.sum(ew, axis=1, keepdims=True)
    z = jnp.where(z == 0.0, 1.0, z)
    avg = s / z
    avg_ref[0] = avg
    stats_ref[0:1, :] += jnp.sum(avg, axis=0, keepdims=True)
    stats_ref[1:2, :] += jnp.sum(avg * avg, axis=0, keepdims=True)

    @pl.when(i == 0)
    def _node_stats():
        xs = jnp.sum(x, axis=0, keepdims=True)
        xss = jnp.sum(x * x, axis=0, keepdims=True)
        stats_ref[pl.ds(2 + 2 * t, 2), :] = jnp.concatenate([xs, xss], axis=0)


def _k2_body(x_ref, avg_ref, wab_ref, wc_ref, bias_ref, h_ref, hstats_ref):
    t = pl.program_id(0)
    i = pl.program_id(1)

    @pl.when(jnp.logical_and(t == 0, i == 0))
    def _init():
        hstats_ref[...] = jnp.zeros_like(hstats_ref)

    x = x_ref[0]
    avg = avg_ref[0]
    h = (jnp.dot(x, wab_ref[0], preferred_element_type=jnp.float32)
         + jnp.dot(avg, wc_ref[...], preferred_element_type=jnp.float32)
         + bias_ref[...])
    h = jnp.maximum(h, 0.0)
    h_ref[0] = h
    hstats_ref[0:1, :] += jnp.sum(h, axis=0, keepdims=True)
    hstats_ref[1:2, :] += jnp.sum(h * h, axis=0, keepdims=True)


def _k3_body(h_ref, w_ref, bias_ref, out_ref):
    h = h_ref[0]
    out = jnp.dot(h, w_ref[...], preferred_element_type=jnp.float32) + bias_ref[...]
    out_ref[0] = jnp.maximum(out, 0.0)


@jax.jit
def kernel(node_data, edge_weights, W0, b0, W1, b1, bn0_g, bn0_b, bn1_g, bn1_b):
    t, n, dh = node_data.shape
    nt1 = n // TILE1
    nt2 = n // TILE2
    m = t * n

    avg, stats = pl.pallas_call(
        _k1_body,
        grid=(t, nt1),
        in_specs=[
            pl.BlockSpec((1, TILE1, n), lambda tt, ii: (tt, ii, 0)),
            pl.BlockSpec((1, n, dh), lambda tt, ii: (tt, 0, 0)),
        ],
        out_specs=[
            pl.BlockSpec((1, TILE1, dh), lambda tt, ii: (tt, ii, 0)),
            pl.BlockSpec((2 + 2 * t, dh), lambda tt, ii: (0, 0)),
        ],
        out_shape=[
            jax.ShapeDtypeStruct((t, n, dh), jnp.float32),
            jax.ShapeDtypeStruct((2 + 2 * t, dh), jnp.float32),
        ],
    )(edge_weights, node_data)

    node_sums = stats[2::2]
    node_sqs = stats[3::2]
    sum_x = jnp.sum(node_sums, axis=0)
    sq_x = jnp.sum(node_sqs, axis=0)
    sum_p = jnp.sum(node_sums[1:], axis=0)
    sq_p = jnp.sum(node_sqs[1:], axis=0)
    sum_a = stats[0]
    sq_a = stats[1]
    m0 = jnp.concatenate([sum_x, sum_p, sum_a]) / m
    v0 = jnp.concatenate([sq_x, sq_p, sq_a]) / m - m0 * m0
    s0 = bn0_g * jax.lax.rsqrt(v0 + 1e-5)
    c0 = bn0_b - m0 * s0
    w0eff = W0 * s0[None, :]
    bias0 = (b0 + W0 @ c0).reshape(1, dh)
    a_t = w0eff[:, :dh].T
    b_t = w0eff[:, dh:2 * dh].T
    c_t = w0eff[:, 2 * dh:].T
    wab = jnp.stack([a_t, a_t + b_t])

    h, hstats = pl.pallas_call(
        _k2_body,
        grid=(t, nt2),
        in_specs=[
            pl.BlockSpec((1, TILE2, dh), lambda tt, ii: (tt, ii, 0)),
            pl.BlockSpec((1, TILE2, dh), lambda tt, ii: (tt, ii, 0)),
            pl.BlockSpec((1, dh, dh), lambda tt, ii: (jnp.minimum(tt, 1), 0, 0)),
            pl.BlockSpec((dh, dh), lambda tt, ii: (0, 0)),
            pl.BlockSpec((1, dh), lambda tt, ii: (0, 0)),
        ],
        out_specs=[
            pl.BlockSpec((1, TILE2, dh), lambda tt, ii: (tt, ii, 0)),
            pl.BlockSpec((2, dh), lambda tt, ii: (0, 0)),
        ],
        out_shape=[
            jax.ShapeDtypeStruct((t, n, dh), jnp.float32),
            jax.ShapeDtypeStruct((2, dh), jnp.float32),
        ],
    )(node_data, avg, wab, c_t, bias0)

    m1 = hstats[0] / m
    v1 = hstats[1] / m - m1 * m1
    s1 = bn1_g * jax.lax.rsqrt(v1 + 1e-5)
    c1 = bn1_b - m1 * s1
    w1eff_t = (W1 * s1[None, :]).T
    bias1 = (b1 + W1 @ c1).reshape(1, dh)

    out = pl.pallas_call(
        _k3_body,
        grid=(t, nt2),
        in_specs=[
            pl.BlockSpec((1, TILE2, dh), lambda tt, ii: (tt, ii, 0)),
            pl.BlockSpec((dh, dh), lambda tt, ii: (0, 0)),
            pl.BlockSpec((1, dh), lambda tt, ii: (0, 0)),
        ],
        out_specs=pl.BlockSpec((1, TILE2, dh), lambda tt, ii: (tt, ii, 0)),
        out_shape=jax.ShapeDtypeStruct((t, n, dh), jnp.float32),
    )(h, w1eff_t, bias1)
    return out
